# baseline (speedup 1.0000x reference)
import functools

import jax
import jax.numpy as jnp
from jax import lax
from jax.experimental import pallas as pl
from jax.experimental.pallas import tpu as pltpu
from jax.experimental.pallas import tpu_sc as plsc

K_NB = 10
N_CLASSES = 1000
BIG_I32 = 2**30


def _topk_body(nq, blk, n_keys, n_blocks,
               q2_ref, qsq_ref, k_ref, od_ref, oi_ref,
               dist_s, rund_s, runi_s):
    i = pl.program_id(0)
    lane16 = lax.broadcasted_iota(jnp.int32, (nq, 16), 1)

    @pl.when(i == 0)
    def _init():
        rund_s[...] = jnp.where(lane16 < K_NB, jnp.inf, -jnp.inf).astype(jnp.float32)
        runi_s[...] = jnp.zeros((nq, 16), jnp.int32)

    q2 = q2_ref[...]
    q_sq = qsq_ref[...]
    k = k_ref[...]
    k_sq = jnp.sum(k * k, axis=1)
    kiota = i * blk + lax.broadcasted_iota(jnp.int32, (blk,), 0)
    k_sq = jnp.where(kiota < n_keys, k_sq, jnp.inf)
    mm = lax.dot_general(q2, k, (((1,), (1,)), ((), ())),
                         preferred_element_type=jnp.float32)
    gidx = i * blk + lax.broadcasted_iota(jnp.int32, (nq, blk), 1)
    dist = (q_sq + k_sq[None, :]) + mm
    dist_s[...] = dist

    thr = jnp.max(rund_s[...], axis=1, keepdims=True)
    bm = jnp.min(dist, axis=1, keepdims=True)
    go = jnp.any(bm < thr)

    @pl.when(go)
    def _merge():
        cnt = jnp.sum((dist < thr).astype(jnp.int32), axis=1)
        rounds = jnp.max(cnt)

        for j in range(K_NB):
            @pl.when(j < rounds)
            def _round():
                d = dist_s[...]
                m = jnp.min(d, axis=1, keepdims=True)
                am = jnp.min(jnp.where(d == m, gidx, BIG_I32),
                             axis=1, keepdims=True)
                dist_s[...] = jnp.where(gidx == am, jnp.inf, d)

                run_d = rund_s[...]
                run_i = runi_s[...]
                rm = jnp.max(run_d, axis=1, keepdims=True)
                slot = jnp.min(jnp.where(run_d == rm, lane16, BIG_I32),
                               axis=1, keepdims=True)
                sel = (lane16 == slot) & (m < rm)
                rund_s[...] = jnp.where(sel, m, run_d)
                runi_s[...] = jnp.where(sel, am, run_i)

    @pl.when(i == n_blocks - 1)
    def _finalize():
        d = jnp.where(lane16 < K_NB, rund_s[...], jnp.inf)
        ii = runi_s[...]
        outd = jnp.zeros((nq, 16), jnp.float32)
        outi = jnp.zeros((nq, 16), jnp.int32)
        for j in range(K_NB):
            m = jnp.min(d, axis=1, keepdims=True)
            ai = jnp.min(jnp.where(d == m, ii, BIG_I32), axis=1, keepdims=True)
            outd = jnp.where(lane16 == j, m, outd)
            outi = jnp.where(lane16 == j, ai, outi)
            d = jnp.where((d == m) & (ii == ai), jnp.inf, d)
        od_ref[...] = outd
        oi_ref[...] = outi


def _topk_call(queries, keys, blk=8192):
    nq = queries.shape[0]
    n_keys = keys.shape[0]
    n_blocks = -(-n_keys // blk)
    q2 = queries * -2.0
    q_sq = jnp.sum(queries * queries, axis=1, keepdims=True)
    body = functools.partial(_topk_body, nq, blk, n_keys, n_blocks)
    od, oi = pl.pallas_call(
        body,
        grid=(n_blocks,),
        in_specs=[
            pl.BlockSpec((nq, 64), lambda i: (0, 0)),
            pl.BlockSpec((nq, 1), lambda i: (0, 0)),
            pl.BlockSpec((blk, 64), lambda i: (i, 0)),
        ],
        out_specs=[
            pl.BlockSpec((nq, 16), lambda i: (0, 0)),
            pl.BlockSpec((nq, 16), lambda i: (0, 0)),
        ],
        out_shape=[
            jax.ShapeDtypeStruct((nq, 16), jnp.float32),
            jax.ShapeDtypeStruct((nq, 16), jnp.int32),
        ],
        scratch_shapes=[
            pltpu.VMEM((nq, blk), jnp.float32),
            pltpu.VMEM((nq, 16), jnp.float32),
            pltpu.VMEM((nq, 16), jnp.int32),
        ],
    )(q2, q_sq, keys)
    return od, oi


def _gather_call(labels, idx_flat):
    n_idx = idx_flat.shape[0]
    info = plsc.get_sparse_core_info()
    nc, ns = info.num_cores, info.num_subcores
    nw = nc * ns
    bpw = n_idx // nw
    mesh = plsc.VectorSubcoreMesh(core_axis_name="c", subcore_axis_name="s",
                                  num_cores=nc)

    @functools.partial(
        pl.kernel, mesh=mesh,
        out_type=jax.ShapeDtypeStruct((n_idx,), jnp.int32),
        scratch_types=[
            pltpu.VMEM((bpw,), jnp.int32),
            pltpu.VMEM((bpw,), jnp.int32),
            pltpu.SemaphoreType.DMA,
        ],
    )
    def _gather(tab_hbm, idx_hbm, out_hbm, idx_v, out_v, sem):
        wid = lax.axis_index("s") * nc + lax.axis_index("c")
        base = wid * bpw
        pltpu.sync_copy(idx_hbm.at[pl.ds(base, bpw)], idx_v)
        pltpu.async_copy(tab_hbm.at[idx_v], out_v, sem).wait()
        pltpu.sync_copy(out_v, out_hbm.at[pl.ds(base, bpw)])

    return _gather(labels, idx_flat)


def _vote_body(nq, nl_ref, o_ref):
    ciota = lax.broadcasted_iota(jnp.int32, (nq, 1024), 1)
    counts = jnp.zeros((nq, 1024), jnp.int32)
    for j in range(K_NB):
        counts = counts + (ciota == nl_ref[:, j:j + 1]).astype(jnp.int32)
    cmax = jnp.max(counts, axis=1, keepdims=True)
    o_ref[...] = jnp.min(jnp.where(counts == cmax, ciota, BIG_I32),
                         axis=1, keepdims=True)


def _vote_call(nl):
    nq = nl.shape[0]
    return pl.pallas_call(
        functools.partial(_vote_body, nq),
        out_shape=jax.ShapeDtypeStruct((nq, 1), jnp.int32),
    )(nl)


def kernel(queries, keys, labels):
    nq = queries.shape[0]
    labels = labels.astype(jnp.int32)
    top_d16, top_i16 = _topk_call(queries, keys)
    top_d = top_d16[:, :K_NB]
    top_i = top_i16[:, :K_NB]

    info = plsc.get_sparse_core_info()
    nw16 = 16 * info.num_cores * info.num_subcores
    n_idx = nq * K_NB
    pad = -n_idx % nw16
    idx_flat = jnp.concatenate(
        [top_i.reshape(-1), jnp.zeros((pad,), jnp.int32)])
    g = _gather_call(labels, idx_flat)
    nl = g[:n_idx].reshape(nq, K_NB)

    pred = _vote_call(nl).reshape(nq)
    return pred, top_d, nl


# speedup vs baseline: 1.0418x; 1.0418x over previous
import functools

import jax
import jax.numpy as jnp
from jax import lax
from jax.experimental import pallas as pl
from jax.experimental.pallas import tpu as pltpu
from jax.experimental.pallas import tpu_sc as plsc

K_NB = 10
N_CLASSES = 1000
BIG_I32 = 2**30


def _topk_body(nq, blk, n_keys, n_blocks,
               q_ref, k_ref, od_ref, oi_ref,
               dist_s, rund_s, runi_s):
    i = pl.program_id(0)
    lane16 = lax.broadcasted_iota(jnp.int32, (nq, 16), 1)

    @pl.when(i == 0)
    def _init():
        rund_s[...] = jnp.where(lane16 < K_NB, jnp.inf, -jnp.inf).astype(jnp.float32)
        runi_s[...] = jnp.zeros((nq, 16), jnp.int32)

    q = q_ref[...]
    k = k_ref[...]
    q_sq = jnp.sum(q * q, axis=1, keepdims=True)
    k_sq = jnp.sum(k * k, axis=1)
    kiota = i * blk + lax.broadcasted_iota(jnp.int32, (blk,), 0)
    k_sq = jnp.where(kiota < n_keys, k_sq, jnp.inf)
    mm = lax.dot_general(q, k, (((1,), (1,)), ((), ())),
                         preferred_element_type=jnp.float32)
    gidx = i * blk + lax.broadcasted_iota(jnp.int32, (nq, blk), 1)
    dist = (q_sq + k_sq[None, :]) - 2.0 * mm
    dist_s[...] = dist

    thr = jnp.max(rund_s[...], axis=1, keepdims=True)
    bm = jnp.min(dist, axis=1, keepdims=True)
    go = jnp.any(bm < thr)

    @pl.when(go)
    def _merge():
        cnt = jnp.sum((dist < thr).astype(jnp.int32), axis=1)
        rounds = jnp.max(cnt)

        for j in range(K_NB):
            @pl.when(j < rounds)
            def _round():
                d = dist_s[...]
                m = jnp.min(d, axis=1, keepdims=True)
                am = jnp.min(jnp.where(d == m, gidx, BIG_I32),
                             axis=1, keepdims=True)
                dist_s[...] = jnp.where(gidx == am, jnp.inf, d)

                run_d = rund_s[...]
                run_i = runi_s[...]
                rm = jnp.max(run_d, axis=1, keepdims=True)
                slot = jnp.min(jnp.where(run_d == rm, lane16, BIG_I32),
                               axis=1, keepdims=True)
                sel = (lane16 == slot) & (m < rm)
                rund_s[...] = jnp.where(sel, m, run_d)
                runi_s[...] = jnp.where(sel, am, run_i)

    @pl.when(i == n_blocks - 1)
    def _finalize():
        d = jnp.where(lane16 < K_NB, rund_s[...], jnp.inf)
        ii = runi_s[...]
        outd = jnp.zeros((nq, 16), jnp.float32)
        outi = jnp.zeros((nq, 16), jnp.int32)
        for j in range(K_NB):
            m = jnp.min(d, axis=1, keepdims=True)
            ai = jnp.min(jnp.where(d == m, ii, BIG_I32), axis=1, keepdims=True)
            outd = jnp.where(lane16 == j, m, outd)
            outi = jnp.where(lane16 == j, ai, outi)
            d = jnp.where((d == m) & (ii == ai), jnp.inf, d)
        od_ref[...] = outd
        oi_ref[...] = outi


def _topk_call(queries, keys, blk=8192):
    nq = queries.shape[0]
    n_keys = keys.shape[0]
    n_blocks = -(-n_keys // blk)
    body = functools.partial(_topk_body, nq, blk, n_keys, n_blocks)
    od, oi = pl.pallas_call(
        body,
        grid=(n_blocks,),
        in_specs=[
            pl.BlockSpec((nq, 64), lambda i: (0, 0)),
            pl.BlockSpec((blk, 64), lambda i: (i, 0)),
        ],
        out_specs=[
            pl.BlockSpec((nq, 16), lambda i: (0, 0)),
            pl.BlockSpec((nq, 16), lambda i: (0, 0)),
        ],
        out_shape=[
            jax.ShapeDtypeStruct((nq, 16), jnp.float32),
            jax.ShapeDtypeStruct((nq, 16), jnp.int32),
        ],
        scratch_shapes=[
            pltpu.VMEM((nq, blk), jnp.float32),
            pltpu.VMEM((nq, 16), jnp.float32),
            pltpu.VMEM((nq, 16), jnp.int32),
        ],
    )(queries, keys)
    return od, oi


def _gather_call(labels, idx_flat):
    n_idx = idx_flat.shape[0]
    info = plsc.get_sparse_core_info()
    nc, ns = info.num_cores, info.num_subcores
    nw = nc * ns
    bpw = n_idx // nw
    mesh = plsc.VectorSubcoreMesh(core_axis_name="c", subcore_axis_name="s",
                                  num_cores=nc)

    @functools.partial(
        pl.kernel, mesh=mesh,
        out_type=jax.ShapeDtypeStruct((n_idx,), jnp.int32),
        scratch_types=[
            pltpu.VMEM((bpw,), jnp.int32),
            pltpu.VMEM((bpw,), jnp.int32),
            pltpu.SemaphoreType.DMA,
        ],
    )
    def _gather(tab_hbm, idx_hbm, out_hbm, idx_v, out_v, sem):
        wid = lax.axis_index("s") * nc + lax.axis_index("c")
        base = wid * bpw
        pltpu.sync_copy(idx_hbm.at[pl.ds(base, bpw)], idx_v)
        pltpu.async_copy(tab_hbm.at[idx_v], out_v, sem).wait()
        pltpu.sync_copy(out_v, out_hbm.at[pl.ds(base, bpw)])

    return _gather(labels, idx_flat)


def _vote_body(nq, nl_ref, o_ref):
    ciota = lax.broadcasted_iota(jnp.int32, (nq, 1024), 1)
    counts = jnp.zeros((nq, 1024), jnp.int32)
    for j in range(K_NB):
        counts = counts + (ciota == nl_ref[:, j:j + 1]).astype(jnp.int32)
    cmax = jnp.max(counts, axis=1, keepdims=True)
    o_ref[...] = jnp.min(jnp.where(counts == cmax, ciota, BIG_I32),
                         axis=1, keepdims=True)


def _vote_call(nl):
    nq = nl.shape[0]
    return pl.pallas_call(
        functools.partial(_vote_body, nq),
        out_shape=jax.ShapeDtypeStruct((nq, 1), jnp.int32),
    )(nl)


def kernel(queries, keys, labels):
    nq = queries.shape[0]
    labels = labels.astype(jnp.int32)
    top_d16, top_i16 = _topk_call(queries, keys)
    top_d = top_d16[:, :K_NB]
    top_i = top_i16[:, :K_NB]

    info = plsc.get_sparse_core_info()
    nw16 = 16 * info.num_cores * info.num_subcores
    n_idx = nq * K_NB
    pad = -n_idx % nw16
    idx_flat = jnp.concatenate(
        [top_i.reshape(-1), jnp.zeros((pad,), jnp.int32)])
    g = _gather_call(labels, idx_flat)
    nl = g[:n_idx].reshape(nq, K_NB)

    pred = _vote_call(nl).reshape(nq)
    return pred, top_d, nl
